# speedup vs baseline: 3.9281x; 3.9281x over previous
import jax
import jax.numpy as jnp
from jax.experimental import pallas as pl

OUT_F = 4096
N_POS = 3072
LEVELS = 256
BATCH = 8
DT = 512


def _body(x_ref, pw_ref, vw_ref, out_ref):
    idx = jnp.clip(jnp.round(x_ref[...] * (LEVELS - 1)), 0, LEVELS - 1).astype(
        jnp.int32
    )
    pw = pw_ref[...]
    vw = vw_ref[...]
    lv = jax.lax.broadcasted_iota(jnp.int32, (LEVELS, N_POS), 0)
    for b in range(BATCH):
        h = (lv == idx[b : b + 1, :]).astype(jnp.bfloat16)
        g = jax.lax.dot_general(
            h, pw, (((1,), (0,)), ((), ())), preferred_element_type=jnp.float32
        )
        ms = jnp.sum(vw * g, axis=0, keepdims=True)
        out_ref[b : b + 1, :] = jnp.where(ms > 0, 1.0, -1.0)


def kernel(x, position_weight, value_weight):
    xf = x.reshape(BATCH, N_POS)
    pw = position_weight.astype(jnp.bfloat16)
    return pl.pallas_call(
        _body,
        grid=(OUT_F // DT,),
        in_specs=[
            pl.BlockSpec((BATCH, N_POS), lambda j: (0, 0)),
            pl.BlockSpec((N_POS, DT), lambda j: (0, j)),
            pl.BlockSpec((LEVELS, DT), lambda j: (0, j)),
        ],
        out_specs=pl.BlockSpec((BATCH, DT), lambda j: (0, j)),
        out_shape=jax.ShapeDtypeStruct((BATCH, OUT_F), jnp.float32),
    )(xf, pw, value_weight)
